# Initial kernel scaffold; baseline (speedup 1.0000x reference)
import jax
import jax.numpy as jnp
from jax.experimental import pallas as pl


def kernel(id, start, emb0, emb1, emb2, gamma, beta):
    raise NotImplementedError("write your pallas kernel here")


# speedup vs baseline: 3.8082x; 3.8082x over previous
import functools

import jax
import jax.numpy as jnp
from jax.experimental import pallas as pl
from jax.experimental.pallas import tpu as pltpu
from jax.experimental.pallas import tpu_sc as plsc

_NUMBERS = (99991, 100003, 100019)
_D = 64
_W = 128


def _mod_body(id_ref, start_ref, r0_ref, r1_ref, r2_ref):
    idx = id_ref[...] + start_ref[...]
    r0_ref[...] = idx % _NUMBERS[0]
    r1_ref[...] = idx % _NUMBERS[1]
    r2_ref[...] = idx % _NUMBERS[2]


def _mod_hashes(id, start):
    B, L = id.shape
    RB = 512
    out = jax.ShapeDtypeStruct((B, L), jnp.int32)
    return pl.pallas_call(
        _mod_body,
        grid=(B // RB,),
        in_specs=[
            pl.BlockSpec((RB, L), lambda i: (i, 0)),
            pl.BlockSpec((RB, 1), lambda i: (i, 0)),
        ],
        out_specs=[
            pl.BlockSpec((RB, L), lambda i: (i, 0)),
            pl.BlockSpec((RB, L), lambda i: (i, 0)),
            pl.BlockSpec((RB, L), lambda i: (i, 0)),
        ],
        out_shape=[out, out, out],
    )(id, start)


def _sc_gather_sum(e0, e1, e2, r0, r1, r2):
    ntok = r0.shape[1]
    mesh = plsc.VectorSubcoreMesh(core_axis_name="c", subcore_axis_name="s")

    @functools.partial(
        pl.kernel,
        out_type=jax.ShapeDtypeStruct((ntok, _D), jnp.float32),
        mesh=mesh,
        compiler_params=pltpu.CompilerParams(use_tc_tiling_on_sc=False),
        scratch_types=[
            pltpu.VMEM((_W, _D), jnp.float32),
            pltpu.VMEM((_W, _D), jnp.float32),
            pltpu.SemaphoreType.DMA,
            pltpu.SemaphoreType.DMA,
        ],
    )
    def k(e0_hbm, e1_hbm, e2_hbm, r0_hbm, r1_hbm, r2_hbm, o_hbm, g1, g2, sem1, sem2):
        def body(r0_v, r1_v, r2_v, o_v):
            cp1 = pltpu.async_copy(e1_hbm.at[r1_v.at[0]], g1, sem1)
            cp2 = pltpu.async_copy(e2_hbm.at[r2_v.at[0]], g2, sem2)
            pltpu.sync_copy(e0_hbm.at[r0_v.at[0]], o_v)
            cp1.wait()
            cp2.wait()

            @pl.loop(0, _W)
            def _(i):
                @pl.loop(0, _D, step=16)
                def _(j):
                    sl = (i, pl.ds(j, 16))
                    o_v[sl] = o_v[sl] + g1[sl] + g2[sl]

        pltpu.emit_pipeline(
            body,
            grid=(ntok // _W,),
            in_specs=[
                pl.BlockSpec((1, _W), lambda i: (0, i)),
                pl.BlockSpec((1, _W), lambda i: (0, i)),
                pl.BlockSpec((1, _W), lambda i: (0, i)),
            ],
            out_specs=[pl.BlockSpec((_W, _D), lambda i: (i, 0))],
            core_axis_name=("c", "s"),
            dimension_semantics=(pltpu.PARALLEL,),
        )(r0_hbm, r1_hbm, r2_hbm, o_hbm)

    return k(e0, e1, e2, r0, r1, r2)


def _ln_body(pe_ref, g_ref, b_ref, o_ref):
    x = pe_ref[...]
    mu = jnp.mean(x, axis=-1, keepdims=True)
    var = jnp.mean((x - mu) ** 2, axis=-1, keepdims=True)
    o_ref[...] = (x - mu) * jax.lax.rsqrt(var + 1e-5) * g_ref[...] + b_ref[...]


def _layer_norm(pe, gamma, beta):
    ntok = pe.shape[0]
    TB = 1024
    return pl.pallas_call(
        _ln_body,
        grid=(ntok // TB,),
        in_specs=[
            pl.BlockSpec((TB, _D), lambda i: (i, 0)),
            pl.BlockSpec((1, _D), lambda i: (0, 0)),
            pl.BlockSpec((1, _D), lambda i: (0, 0)),
        ],
        out_specs=pl.BlockSpec((TB, _D), lambda i: (i, 0)),
        out_shape=jax.ShapeDtypeStruct((ntok, _D), jnp.float32),
    )(pe, gamma, beta)


def kernel(id, start, emb0, emb1, emb2, gamma, beta):
    B, L = id.shape
    ntok = B * L
    r0, r1, r2 = _mod_hashes(id, start)
    pe = _sc_gather_sum(
        emb0, emb1, emb2,
        r0.reshape(1, ntok), r1.reshape(1, ntok), r2.reshape(1, ntok),
    )
    out = _layer_norm(pe, gamma.reshape(1, _D), beta.reshape(1, _D))
    return out.reshape(B, L, _D)


# speedup vs baseline: 4.1097x; 1.0792x over previous
import functools

import jax
import jax.numpy as jnp
from jax.experimental import pallas as pl
from jax.experimental.pallas import tpu as pltpu
from jax.experimental.pallas import tpu_sc as plsc

_NUMBERS = (99991, 100003, 100019)
_D = 64
_W = 128


def _mod_body(id_ref, start_ref, r0_ref, r1_ref, r2_ref):
    idx = id_ref[...] + start_ref[...]
    r0_ref[...] = idx % _NUMBERS[0]
    r1_ref[...] = idx % _NUMBERS[1]
    r2_ref[...] = idx % _NUMBERS[2]


def _mod_hashes(id, start):
    B, L = id.shape
    RB = 512
    out = jax.ShapeDtypeStruct((B, L), jnp.int32)
    return pl.pallas_call(
        _mod_body,
        grid=(B // RB,),
        in_specs=[
            pl.BlockSpec((RB, L), lambda i: (i, 0)),
            pl.BlockSpec((RB, 1), lambda i: (i, 0)),
        ],
        out_specs=[
            pl.BlockSpec((RB, L), lambda i: (i, 0)),
            pl.BlockSpec((RB, L), lambda i: (i, 0)),
            pl.BlockSpec((RB, L), lambda i: (i, 0)),
        ],
        out_shape=[out, out, out],
    )(id, start)


def _sc_gather_sum(e0, e1, e2, r0, r1, r2):
    ntok = r0.shape[1]
    mesh = plsc.VectorSubcoreMesh(core_axis_name="c", subcore_axis_name="s")

    @functools.partial(
        pl.kernel,
        out_type=jax.ShapeDtypeStruct((ntok, _D), jnp.float32),
        mesh=mesh,
        compiler_params=pltpu.CompilerParams(use_tc_tiling_on_sc=False),
        scratch_types=[
            pltpu.VMEM((_W, _D), jnp.float32),
            pltpu.VMEM((_W, _D), jnp.float32),
            pltpu.SemaphoreType.DMA,
            pltpu.SemaphoreType.DMA,
        ],
    )
    def k(e0_hbm, e1_hbm, e2_hbm, r0_hbm, r1_hbm, r2_hbm, o_hbm, g1, g2, sem1, sem2):
        def body(r0_v, r1_v, r2_v, o_v):
            cp1 = pltpu.async_copy(e1_hbm.at[r1_v.at[0]], g1, sem1)
            cp2 = pltpu.async_copy(e2_hbm.at[r2_v.at[0]], g2, sem2)
            pltpu.sync_copy(e0_hbm.at[r0_v.at[0]], o_v)
            cp1.wait()
            cp2.wait()

            @pl.loop(0, _W)
            def _(i):
                for j in range(0, _D, 16):
                    sl = (i, pl.ds(j, 16))
                    o_v[sl] = o_v[sl] + g1[sl] + g2[sl]

        pltpu.emit_pipeline(
            body,
            grid=(ntok // _W,),
            in_specs=[
                pl.BlockSpec((1, _W), lambda i: (0, i)),
                pl.BlockSpec((1, _W), lambda i: (0, i)),
                pl.BlockSpec((1, _W), lambda i: (0, i)),
            ],
            out_specs=[pl.BlockSpec((_W, _D), lambda i: (i, 0))],
            core_axis_name=("c", "s"),
            dimension_semantics=(pltpu.PARALLEL,),
        )(r0_hbm, r1_hbm, r2_hbm, o_hbm)

    return k(e0, e1, e2, r0, r1, r2)


_LN_ROWS = 16


def _ln_body(pe_ref, g_ref, b_ref, o_ref):
    x = pe_ref[...]
    mu = jnp.mean(x, axis=-1, keepdims=True)
    var = jnp.mean((x - mu) ** 2, axis=-1, keepdims=True)
    y = (x - mu) * jax.lax.rsqrt(var + 1e-5) * g_ref[...] + b_ref[...]
    o_ref[...] = y.reshape(o_ref.shape)


def _layer_norm(pe, gamma, beta, B, L):
    ntok = pe.shape[0]
    TB = _LN_ROWS * L
    return pl.pallas_call(
        _ln_body,
        grid=(ntok // TB,),
        in_specs=[
            pl.BlockSpec((TB, _D), lambda i: (i, 0)),
            pl.BlockSpec((1, _D), lambda i: (0, 0)),
            pl.BlockSpec((1, _D), lambda i: (0, 0)),
        ],
        out_specs=pl.BlockSpec((_LN_ROWS, L, _D), lambda i: (i, 0, 0)),
        out_shape=jax.ShapeDtypeStruct((B, L, _D), jnp.float32),
    )(pe, gamma, beta)


def kernel(id, start, emb0, emb1, emb2, gamma, beta):
    B, L = id.shape
    ntok = B * L
    r0, r1, r2 = _mod_hashes(id, start)
    pe = _sc_gather_sum(
        emb0, emb1, emb2,
        r0.reshape(1, ntok), r1.reshape(1, ntok), r2.reshape(1, ntok),
    )
    return _layer_norm(pe, gamma.reshape(1, _D), beta.reshape(1, _D), B, L)


# speedup vs baseline: 4.5113x; 1.0977x over previous
import functools

import jax
import jax.numpy as jnp
from jax.experimental import pallas as pl
from jax.experimental.pallas import tpu as pltpu
from jax.experimental.pallas import tpu_sc as plsc

_NUMBERS = (99991, 100003, 100019)
_D = 64
_W = 128


def _mod_body(id_ref, start_ref, r0_ref, r1_ref, r2_ref):
    idx = id_ref[...] + start_ref[...]
    r0_ref[...] = idx % _NUMBERS[0]
    r1_ref[...] = idx % _NUMBERS[1]
    r2_ref[...] = idx % _NUMBERS[2]


def _mod_hashes(id, start):
    B, L = id.shape
    RB = 512
    out = jax.ShapeDtypeStruct((B, L), jnp.int32)
    return pl.pallas_call(
        _mod_body,
        grid=(B // RB,),
        in_specs=[
            pl.BlockSpec((RB, L), lambda i: (i, 0)),
            pl.BlockSpec((RB, 1), lambda i: (i, 0)),
        ],
        out_specs=[
            pl.BlockSpec((RB, L), lambda i: (i, 0)),
            pl.BlockSpec((RB, L), lambda i: (i, 0)),
            pl.BlockSpec((RB, L), lambda i: (i, 0)),
        ],
        out_shape=[out, out, out],
    )(id, start)


def _sc_gather_sum(e0, e1, e2, r0, r1, r2):
    ntok = r0.shape[1]
    mesh = plsc.VectorSubcoreMesh(core_axis_name="c", subcore_axis_name="s")

    @functools.partial(
        pl.kernel,
        out_type=jax.ShapeDtypeStruct((ntok // 2, 2 * _D), jnp.float32),
        mesh=mesh,
        compiler_params=pltpu.CompilerParams(use_tc_tiling_on_sc=False),
        scratch_types=[
            pltpu.VMEM((_W, _D), jnp.float32),
            pltpu.VMEM((_W, _D), jnp.float32),
            pltpu.VMEM((_W, _D), jnp.float32),
            pltpu.SemaphoreType.DMA,
            pltpu.SemaphoreType.DMA,
            pltpu.SemaphoreType.DMA,
        ],
    )
    def k(e0_hbm, e1_hbm, e2_hbm, r0_hbm, r1_hbm, r2_hbm, o_hbm, g0, g1, g2,
          sem0, sem1, sem2):
        def body(r0_v, r1_v, r2_v, o_v):
            cp0 = pltpu.async_copy(e0_hbm.at[r0_v.at[0]], g0, sem0)
            cp1 = pltpu.async_copy(e1_hbm.at[r1_v.at[0]], g1, sem1)
            cp2 = pltpu.async_copy(e2_hbm.at[r2_v.at[0]], g2, sem2)
            cp0.wait()
            cp1.wait()
            cp2.wait()

            @pl.loop(0, _W // 2)
            def _(p):
                for t in range(2):
                    i = 2 * p + t
                    for j in range(0, _D, 16):
                        src = (i, pl.ds(j, 16))
                        o_v[p, pl.ds(t * _D + j, 16)] = g0[src] + g1[src] + g2[src]

        pltpu.emit_pipeline(
            body,
            grid=(ntok // _W,),
            in_specs=[
                pl.BlockSpec((1, _W), lambda i: (0, i)),
                pl.BlockSpec((1, _W), lambda i: (0, i)),
                pl.BlockSpec((1, _W), lambda i: (0, i)),
            ],
            out_specs=[pl.BlockSpec((_W // 2, 2 * _D), lambda i: (i, 0))],
            core_axis_name=("c", "s"),
            dimension_semantics=(pltpu.PARALLEL,),
        )(r0_hbm, r1_hbm, r2_hbm, o_hbm)

    return k(e0, e1, e2, r0, r1, r2)


def _ln_body(pe_ref, g_ref, b_ref, o_ref):
    x = pe_ref[...]
    lane = jax.lax.broadcasted_iota(jnp.int32, x.shape, 1)
    left = lane < _D
    xl = jnp.where(left, x, 0.0)
    xx = x * x
    s_all = jnp.sum(x, axis=-1, keepdims=True)
    s_l = jnp.sum(xl, axis=-1, keepdims=True)
    q_all = jnp.sum(xx, axis=-1, keepdims=True)
    q_l = jnp.sum(jnp.where(left, xx, 0.0), axis=-1, keepdims=True)
    mu_l = s_l / _D
    mu_r = (s_all - s_l) / _D
    rs_l = jax.lax.rsqrt(q_l / _D - mu_l * mu_l + 1e-5)
    rs_r = jax.lax.rsqrt((q_all - q_l) / _D - mu_r * mu_r + 1e-5)
    mu = jnp.where(left, mu_l, mu_r)
    rs = jnp.where(left, rs_l, rs_r)
    o_ref[...] = (x - mu) * rs * g_ref[...] + b_ref[...]


def _layer_norm(pe, gamma, beta):
    nrow = pe.shape[0]
    TB = 2048
    return pl.pallas_call(
        _ln_body,
        grid=(nrow // TB,),
        in_specs=[
            pl.BlockSpec((TB, 2 * _D), lambda i: (i, 0)),
            pl.BlockSpec((1, 2 * _D), lambda i: (0, 0)),
            pl.BlockSpec((1, 2 * _D), lambda i: (0, 0)),
        ],
        out_specs=pl.BlockSpec((TB, 2 * _D), lambda i: (i, 0)),
        out_shape=jax.ShapeDtypeStruct((nrow, 2 * _D), jnp.float32),
    )(pe, gamma, beta)


def kernel(id, start, emb0, emb1, emb2, gamma, beta):
    B, L = id.shape
    ntok = B * L
    r0, r1, r2 = _mod_hashes(id, start)
    pe = _sc_gather_sum(
        emb0, emb1, emb2,
        r0.reshape(1, ntok), r1.reshape(1, ntok), r2.reshape(1, ntok),
    )
    g2 = jnp.concatenate([gamma, gamma]).reshape(1, 2 * _D)
    b2 = jnp.concatenate([beta, beta]).reshape(1, 2 * _D)
    out = _layer_norm(pe, g2, b2)
    return out.reshape(B, L, _D)


# speedup vs baseline: 4.5720x; 1.0134x over previous
import functools

import jax
import jax.numpy as jnp
from jax.experimental import pallas as pl
from jax.experimental.pallas import tpu as pltpu
from jax.experimental.pallas import tpu_sc as plsc

_NUMBERS = (99991, 100003, 100019)
_D = 64
_W = 128


def _mod_body(id_ref, start_ref, r0_ref, r1_ref, r2_ref):
    idx = id_ref[...] + start_ref[...]
    r0_ref[...] = idx % _NUMBERS[0]
    r1_ref[...] = idx % _NUMBERS[1]
    r2_ref[...] = idx % _NUMBERS[2]


def _mod_hashes(id, start):
    B, L = id.shape
    RB = 512
    out = jax.ShapeDtypeStruct((B, L), jnp.int32)
    return pl.pallas_call(
        _mod_body,
        grid=(B // RB,),
        in_specs=[
            pl.BlockSpec((RB, L), lambda i: (i, 0)),
            pl.BlockSpec((RB, 1), lambda i: (i, 0)),
        ],
        out_specs=[
            pl.BlockSpec((RB, L), lambda i: (i, 0)),
            pl.BlockSpec((RB, L), lambda i: (i, 0)),
            pl.BlockSpec((RB, L), lambda i: (i, 0)),
        ],
        out_shape=[out, out, out],
    )(id, start)


_BI = 2


def _sc_gather_sum(e0, e1, e2, r0, r1, r2):
    nrow = r0.shape[0]
    ntok = nrow * 128
    W = _BI * 128
    mesh = plsc.VectorSubcoreMesh(core_axis_name="c", subcore_axis_name="s")

    @functools.partial(
        pl.kernel,
        out_type=jax.ShapeDtypeStruct((ntok // 2, 2 * _D), jnp.float32),
        mesh=mesh,
        compiler_params=pltpu.CompilerParams(use_tc_tiling_on_sc=False),
        scratch_types=[
            pltpu.VMEM((W, _D), jnp.float32),
            pltpu.VMEM((W, _D), jnp.float32),
            pltpu.VMEM((W, _D), jnp.float32),
            pltpu.SemaphoreType.DMA,
            pltpu.SemaphoreType.DMA,
            pltpu.SemaphoreType.DMA,
        ],
    )
    def k(e0_hbm, e1_hbm, e2_hbm, r0_hbm, r1_hbm, r2_hbm, o_hbm, g0, g1, g2,
          sem0, sem1, sem2):
        def body(r0_v, r1_v, r2_v, o_v):
            cps = []
            for j in range(_BI):
                dst = pl.ds(j * 128, 128)
                cps.append(pltpu.async_copy(e0_hbm.at[r0_v.at[j]], g0.at[dst], sem0))
                cps.append(pltpu.async_copy(e1_hbm.at[r1_v.at[j]], g1.at[dst], sem1))
                cps.append(pltpu.async_copy(e2_hbm.at[r2_v.at[j]], g2.at[dst], sem2))
            for cp in cps:
                cp.wait()

            @pl.loop(0, W // 2)
            def _(p):
                for t in range(2):
                    i = 2 * p + t
                    for j in range(0, _D, 16):
                        src = (i, pl.ds(j, 16))
                        o_v[p, pl.ds(t * _D + j, 16)] = g0[src] + g1[src] + g2[src]

        pltpu.emit_pipeline(
            body,
            grid=(nrow // _BI,),
            in_specs=[
                pl.BlockSpec((_BI, 128), lambda i: (i, 0)),
                pl.BlockSpec((_BI, 128), lambda i: (i, 0)),
                pl.BlockSpec((_BI, 128), lambda i: (i, 0)),
            ],
            out_specs=[pl.BlockSpec((W // 2, 2 * _D), lambda i: (i, 0))],
            core_axis_name=("c", "s"),
            dimension_semantics=(pltpu.PARALLEL,),
        )(r0_hbm, r1_hbm, r2_hbm, o_hbm)

    return k(e0, e1, e2, r0, r1, r2)


def _ln_body(pe_ref, g_ref, b_ref, o_ref):
    x = pe_ref[...]
    lane = jax.lax.broadcasted_iota(jnp.int32, x.shape, 1)
    left = lane < _D
    xl = jnp.where(left, x, 0.0)
    xx = x * x
    s_all = jnp.sum(x, axis=-1, keepdims=True)
    s_l = jnp.sum(xl, axis=-1, keepdims=True)
    q_all = jnp.sum(xx, axis=-1, keepdims=True)
    q_l = jnp.sum(jnp.where(left, xx, 0.0), axis=-1, keepdims=True)
    mu_l = s_l / _D
    mu_r = (s_all - s_l) / _D
    rs_l = jax.lax.rsqrt(q_l / _D - mu_l * mu_l + 1e-5)
    rs_r = jax.lax.rsqrt((q_all - q_l) / _D - mu_r * mu_r + 1e-5)
    mu = jnp.where(left, mu_l, mu_r)
    rs = jnp.where(left, rs_l, rs_r)
    o_ref[...] = (x - mu) * rs * g_ref[...] + b_ref[...]


def _layer_norm(pe, gamma, beta):
    nrow = pe.shape[0]
    TB = 2048
    return pl.pallas_call(
        _ln_body,
        grid=(nrow // TB,),
        in_specs=[
            pl.BlockSpec((TB, 2 * _D), lambda i: (i, 0)),
            pl.BlockSpec((1, 2 * _D), lambda i: (0, 0)),
            pl.BlockSpec((1, 2 * _D), lambda i: (0, 0)),
        ],
        out_specs=pl.BlockSpec((TB, 2 * _D), lambda i: (i, 0)),
        out_shape=jax.ShapeDtypeStruct((nrow, 2 * _D), jnp.float32),
    )(pe, gamma, beta)


def kernel(id, start, emb0, emb1, emb2, gamma, beta):
    B, L = id.shape
    ntok = B * L
    r0, r1, r2 = _mod_hashes(id, start)
    nr = ntok // 128
    pe = _sc_gather_sum(
        emb0, emb1, emb2,
        r0.reshape(nr, 128), r1.reshape(nr, 128), r2.reshape(nr, 128),
    )
    g2 = jnp.concatenate([gamma, gamma]).reshape(1, 2 * _D)
    b2 = jnp.concatenate([beta, beta]).reshape(1, 2 * _D)
    out = _layer_norm(pe, g2, b2)
    return out.reshape(B, L, _D)


# speedup vs baseline: 5.3069x; 1.1608x over previous
import functools

import jax
import jax.numpy as jnp
from jax.experimental import pallas as pl
from jax.experimental.pallas import tpu as pltpu
from jax.experimental.pallas import tpu_sc as plsc

_NUMBERS = (99991, 100003, 100019)
_D = 64
_W = 128


def _mod_body(id_ref, start_ref, r0_ref, r1_ref, r2_ref):
    idx = id_ref[...] + start_ref[...]
    r0_ref[...] = idx % _NUMBERS[0]
    r1_ref[...] = idx % _NUMBERS[1]
    r2_ref[...] = idx % _NUMBERS[2]


def _mod_hashes(id, start):
    B, L = id.shape
    RB = 512
    out = jax.ShapeDtypeStruct((B, L), jnp.int32)
    return pl.pallas_call(
        _mod_body,
        grid=(B // RB,),
        in_specs=[
            pl.BlockSpec((RB, L), lambda i: (i, 0)),
            pl.BlockSpec((RB, 1), lambda i: (i, 0)),
        ],
        out_specs=[
            pl.BlockSpec((RB, L), lambda i: (i, 0)),
            pl.BlockSpec((RB, L), lambda i: (i, 0)),
            pl.BlockSpec((RB, L), lambda i: (i, 0)),
        ],
        out_shape=[out, out, out],
    )(id, start)


_NW = 32


def _sc_gather_sum(e0, e1, e2, r0, r1, r2):
    nrow = r0.shape[0]
    ntok = nrow * 128
    SP = nrow // _NW
    mesh = plsc.VectorSubcoreMesh(core_axis_name="c", subcore_axis_name="s")

    @functools.partial(
        pl.kernel,
        out_type=jax.ShapeDtypeStruct((ntok // 2, 2 * _D), jnp.float32),
        mesh=mesh,
        compiler_params=pltpu.CompilerParams(use_tc_tiling_on_sc=False),
        scratch_types=[
            pltpu.VMEM((128, _D), jnp.float32),
            pltpu.VMEM((128, _D), jnp.float32),
            pltpu.VMEM((128, _D), jnp.float32),
            pltpu.VMEM((128, _D), jnp.float32),
            pltpu.VMEM((128, _D), jnp.float32),
            pltpu.VMEM((128, _D), jnp.float32),
            pltpu.SMEM((1,), jnp.int32),
            pltpu.SemaphoreType.DMA,
            pltpu.SemaphoreType.DMA,
            pltpu.SemaphoreType.DMA,
        ],
    )
    def k(e0_hbm, e1_hbm, e2_hbm, r0_hbm, r1_hbm, r2_hbm, o_hbm,
          ga0, ga1, ga2, gb0, gb1, gb2, cnt, sem0, sem1, sem2):
        cnt[0] = 0

        def _sum_into(o_v, g0, g1, g2):
            @pl.loop(0, 64)
            def _(p):
                for t in range(2):
                    i = 2 * p + t
                    for j in range(0, _D, 16):
                        src = (i, pl.ds(j, 16))
                        o_v[p, pl.ds(t * _D + j, 16)] = g0[src] + g1[src] + g2[src]

        def body(r0_v, r1_v, r2_v, o_v):
            s = cnt[0]
            par = jax.lax.rem(s, 2)

            @pl.when(jnp.logical_and(s < SP, par == 0))
            def _():
                pltpu.async_copy(e0_hbm.at[r0_v.at[0]], ga0, sem0)
                pltpu.async_copy(e1_hbm.at[r1_v.at[0]], ga1, sem1)
                pltpu.async_copy(e2_hbm.at[r2_v.at[0]], ga2, sem2)

            @pl.when(jnp.logical_and(s < SP, par == 1))
            def _():
                pltpu.async_copy(e0_hbm.at[r0_v.at[0]], gb0, sem0)
                pltpu.async_copy(e1_hbm.at[r1_v.at[0]], gb1, sem1)
                pltpu.async_copy(e2_hbm.at[r2_v.at[0]], gb2, sem2)

            @pl.when(s > 0)
            def _():
                pltpu.make_async_copy(e0_hbm.at[pl.ds(0, 128)], ga0, sem0).wait()
                pltpu.make_async_copy(e1_hbm.at[pl.ds(0, 128)], ga1, sem1).wait()
                pltpu.make_async_copy(e2_hbm.at[pl.ds(0, 128)], ga2, sem2).wait()

            @pl.when(jnp.logical_and(s > 0, par == 1))
            def _():
                _sum_into(o_v, ga0, ga1, ga2)

            @pl.when(jnp.logical_and(s > 0, par == 0))
            def _():
                _sum_into(o_v, gb0, gb1, gb2)

            cnt[0] = s + 1

        pltpu.emit_pipeline(
            body,
            grid=(_NW, SP + 1),
            in_specs=[
                pl.BlockSpec((1, 128), lambda w, s: (w * SP + jnp.minimum(s, SP - 1), 0)),
                pl.BlockSpec((1, 128), lambda w, s: (w * SP + jnp.minimum(s, SP - 1), 0)),
                pl.BlockSpec((1, 128), lambda w, s: (w * SP + jnp.minimum(s, SP - 1), 0)),
            ],
            out_specs=[
                pl.BlockSpec((64, 2 * _D), lambda w, s: (w * SP + jnp.maximum(s - 1, 0), 0)),
            ],
            core_axis_name=("c", "s"),
            dimension_semantics=(pltpu.PARALLEL, pltpu.ARBITRARY),
        )(r0_hbm, r1_hbm, r2_hbm, o_hbm)

    return k(e0, e1, e2, r0, r1, r2)


def _ln_body(pe_ref, g_ref, b_ref, o_ref):
    x = pe_ref[...]
    lane = jax.lax.broadcasted_iota(jnp.int32, x.shape, 1)
    left = lane < _D
    xl = jnp.where(left, x, 0.0)
    xx = x * x
    s_all = jnp.sum(x, axis=-1, keepdims=True)
    s_l = jnp.sum(xl, axis=-1, keepdims=True)
    q_all = jnp.sum(xx, axis=-1, keepdims=True)
    q_l = jnp.sum(jnp.where(left, xx, 0.0), axis=-1, keepdims=True)
    mu_l = s_l / _D
    mu_r = (s_all - s_l) / _D
    rs_l = jax.lax.rsqrt(q_l / _D - mu_l * mu_l + 1e-5)
    rs_r = jax.lax.rsqrt((q_all - q_l) / _D - mu_r * mu_r + 1e-5)
    mu = jnp.where(left, mu_l, mu_r)
    rs = jnp.where(left, rs_l, rs_r)
    o_ref[...] = (x - mu) * rs * g_ref[...] + b_ref[...]


def _layer_norm(pe, gamma, beta):
    nrow = pe.shape[0]
    TB = 2048
    return pl.pallas_call(
        _ln_body,
        grid=(nrow // TB,),
        in_specs=[
            pl.BlockSpec((TB, 2 * _D), lambda i: (i, 0)),
            pl.BlockSpec((1, 2 * _D), lambda i: (0, 0)),
            pl.BlockSpec((1, 2 * _D), lambda i: (0, 0)),
        ],
        out_specs=pl.BlockSpec((TB, 2 * _D), lambda i: (i, 0)),
        out_shape=jax.ShapeDtypeStruct((nrow, 2 * _D), jnp.float32),
    )(pe, gamma, beta)


def kernel(id, start, emb0, emb1, emb2, gamma, beta):
    B, L = id.shape
    ntok = B * L
    r0, r1, r2 = _mod_hashes(id, start)
    nr = ntok // 128
    pe = _sc_gather_sum(
        emb0, emb1, emb2,
        r0.reshape(nr, 128), r1.reshape(nr, 128), r2.reshape(nr, 128),
    )
    g2 = jnp.concatenate([gamma, gamma]).reshape(1, 2 * _D)
    b2 = jnp.concatenate([beta, beta]).reshape(1, 2 * _D)
    out = _layer_norm(pe, g2, b2)
    return out.reshape(B, L, _D)
